# Initial kernel scaffold; baseline (speedup 1.0000x reference)
import jax
import jax.numpy as jnp
from jax.experimental import pallas as pl


def kernel(inputs, W, gamma, beta):
    raise NotImplementedError("write your pallas kernel here")


# speedup vs baseline: 8.9091x; 8.9091x over previous
import functools

import jax
import jax.numpy as jnp
from jax.experimental import pallas as pl

K = 20
B, C, N = 8, 32, 2048
O = 64
RT = 256
NT = N // RT
CNT = float(B * N * K)


def _select_kernel(xt_ref, xc_ref, wt_ref, rmax_ref, rmin_ref, stats_ref):
    b = pl.program_id(0)
    t = pl.program_id(1)
    xb = xt_ref[0]
    xc = xc_ref[0]
    tile = xt_ref[0, pl.ds(t * RT, RT), :]

    sq = jnp.sum(xc * xc, axis=0, keepdims=True)
    P = 2.0 * jnp.dot(tile, xc, preferred_element_type=jnp.float32) - sq

    w1t = wt_ref[:C, :]
    w2t = wt_ref[C:, :]
    ub = jnp.dot(xb, w1t, preferred_element_type=jnp.float32)
    u2b = ub * ub
    vt = jnp.dot(tile, w2t - w1t, preferred_element_type=jnp.float32)

    msel = jnp.zeros((RT, N), jnp.float32)
    mmax = jnp.full((RT, O), -jnp.inf, jnp.float32)
    mmin = jnp.full((RT, O), jnp.inf, jnp.float32)
    for _ in range(K):
        rm = jnp.max(P, axis=1, keepdims=True)
        h = P >= rm
        hf = h.astype(jnp.float32)
        U = jnp.dot(hf, ub, preferred_element_type=jnp.float32)
        mmax = jnp.maximum(mmax, U)
        mmin = jnp.minimum(mmin, U)
        msel = msel + hf
        P = jnp.where(h, -jnp.inf, P)

    s1 = jnp.dot(msel, ub, preferred_element_type=jnp.float32)
    s2 = jnp.dot(msel, u2b, preferred_element_type=jnp.float32)

    rmax_ref[0] = vt + mmax
    rmin_ref[0] = vt + mmin

    a = jnp.sum(s1 + float(K) * vt, axis=0, keepdims=True)
    q = jnp.sum(s2 + 2.0 * vt * s1 + float(K) * vt * vt, axis=0,
                keepdims=True)
    acc = jnp.concatenate([a, q, jnp.zeros((6, O), jnp.float32)], axis=0)

    @pl.when(jnp.logical_and(b == 0, t == 0))
    def _init():
        stats_ref[...] = acc

    @pl.when(jnp.logical_not(jnp.logical_and(b == 0, t == 0)))
    def _accum():
        stats_ref[...] += acc


def _finalize_kernel(rmax_ref, rmin_ref, stats_ref, g_ref, be_ref, out_ref):
    a = stats_ref[0:1, :]
    q = stats_ref[1:2, :]
    mean = a / CNT
    var = q / CNT - mean * mean
    scale = g_ref[...] * jax.lax.rsqrt(var + 1e-5)
    shift = be_ref[...] - mean * scale
    raw = jnp.where(scale >= 0.0, rmax_ref[0], rmin_ref[0])
    y = raw * scale + shift
    y = jnp.where(y >= 0.0, y, 0.2 * y)
    out_ref[0] = y.T


@jax.jit
def kernel(inputs, W, gamma, beta):
    xt = jnp.transpose(inputs, (0, 2, 1))
    wt = jnp.transpose(W, (1, 0))

    rmax, rmin, stats = pl.pallas_call(
        _select_kernel,
        grid=(B, NT),
        in_specs=[
            pl.BlockSpec((1, N, C), lambda b, t: (b, 0, 0)),
            pl.BlockSpec((1, C, N), lambda b, t: (b, 0, 0)),
            pl.BlockSpec((2 * C, O), lambda b, t: (0, 0)),
        ],
        out_specs=[
            pl.BlockSpec((1, RT, O), lambda b, t: (b, t, 0)),
            pl.BlockSpec((1, RT, O), lambda b, t: (b, t, 0)),
            pl.BlockSpec((8, O), lambda b, t: (0, 0)),
        ],
        out_shape=[
            jax.ShapeDtypeStruct((B, N, O), jnp.float32),
            jax.ShapeDtypeStruct((B, N, O), jnp.float32),
            jax.ShapeDtypeStruct((8, O), jnp.float32),
        ],
    )(xt, inputs, wt)

    out = pl.pallas_call(
        _finalize_kernel,
        grid=(B, NT),
        in_specs=[
            pl.BlockSpec((1, RT, O), lambda b, t: (b, t, 0)),
            pl.BlockSpec((1, RT, O), lambda b, t: (b, t, 0)),
            pl.BlockSpec((8, O), lambda b, t: (0, 0)),
            pl.BlockSpec((1, O), lambda b, t: (0, 0)),
            pl.BlockSpec((1, O), lambda b, t: (0, 0)),
        ],
        out_specs=pl.BlockSpec((1, O, RT), lambda b, t: (b, 0, t)),
        out_shape=jax.ShapeDtypeStruct((B, O, N), jnp.float32),
    )(rmax, rmin, stats, gamma.reshape(1, O), beta.reshape(1, O))
    return out
